# Initial kernel scaffold; baseline (speedup 1.0000x reference)
import jax
import jax.numpy as jnp
from jax.experimental import pallas as pl


def kernel(boxes, scores):
    raise NotImplementedError("write your pallas kernel here")


# speedup vs baseline: 28.1037x; 28.1037x over previous
import functools

import jax
import jax.numpy as jnp
from jax import lax
from jax.experimental import pallas as pl
from jax.experimental.pallas import tpu as pltpu

SCORE_THRESHOLD = 0.05
IOU_THRESHOLD = 0.5
MAX_DETECTIONS = 100
NEG = -1e30

_N = 5000
_C = 80
_NPAD = 5120
_R = 40
_L = 128
_BIG_I32 = 2**30


def _nms_body(scores_t_ref, boxes3_ref, coords_ref,
              out_b_ref, out_s_ref, out_l_ref, out_v_ref,
              s3_ref, classmax_ref, minb_ref, idx_ref, areas_ref):
    ridx = lax.broadcasted_iota(jnp.int32, (_R, _L), 0)
    lidx = lax.broadcasted_iota(jnp.int32, (_R, _L), 1)
    idx_ref[...] = ridx * _L + lidx
    y1a = coords_ref[0]
    x1a = coords_ref[1]
    y2a = coords_ref[2]
    x2a = coords_ref[3]
    areas_ref[...] = (y2a - y1a) * (x2a - x1a)

    classmax_ref[...] = jnp.full((1, _L), NEG, jnp.float32)
    minb_ref[...] = jnp.zeros((1, _L), jnp.int32)
    lane = lax.broadcasted_iota(jnp.int32, (1, _L), 1)

    def init_c(c, _):
        raw = scores_t_ref[c]
        row = jnp.where(raw > SCORE_THRESHOLD, raw, NEG)
        s3_ref[c] = row
        m_c = jnp.max(row)
        eq = row == m_c
        mb = jnp.min(jnp.where(eq, idx_ref[...], _BIG_I32))
        classmax_ref[...] = jnp.where(lane == c, m_c, classmax_ref[...])
        minb_ref[...] = jnp.where(lane == c, mb, minb_ref[...])
        return 0

    lax.fori_loop(0, _C, init_c, 0)

    def step(i, cnt):
        cm = classmax_ref[...]
        m = jnp.max(cm)
        flatcand = jnp.where(cm == m, minb_ref[...] * _C + lane, _BIG_I32)
        flat = jnp.min(flatcand)
        c_star = flat % _C
        b_star = flat // _C
        valid = m > SCORE_THRESHOLD

        box = boxes3_ref[b_star]
        by1 = box[0:1, 0:1]
        bx1 = box[0:1, 1:2]
        by2 = box[0:1, 2:3]
        bx2 = box[0:1, 3:4]

        y1 = jnp.maximum(by1, coords_ref[0])
        x1 = jnp.maximum(bx1, coords_ref[1])
        y2 = jnp.minimum(by2, coords_ref[2])
        x2 = jnp.minimum(bx2, coords_ref[3])
        inter = jnp.maximum(y2 - y1, 0.0) * jnp.maximum(x2 - x1, 0.0)
        a1 = (by2 - by1) * (bx2 - bx1)
        iou = inter / jnp.maximum(a1 + areas_ref[...] - inter, 1e-9)
        suppress = (iou > IOU_THRESHOLD) | (idx_ref[...] == b_star)

        row = s3_ref[c_star]
        row_new = jnp.where(suppress, NEG, row)
        s3_ref[c_star] = row_new

        m_c = jnp.max(row_new)
        eq = row_new == m_c
        mb = jnp.min(jnp.where(eq, idx_ref[...], _BIG_I32))
        classmax_ref[...] = jnp.where(lane == c_star, m_c, classmax_ref[...])
        minb_ref[...] = jnp.where(lane == c_star, mb, minb_ref[...])

        out_b_ref[i] = jnp.where(valid, box, 0.0)
        out_s_ref[i] = jnp.where(valid, m, 0.0).reshape(1, 1)
        out_l_ref[i] = jnp.where(valid, c_star, 0).astype(jnp.int32).reshape(1, 1)
        return cnt + valid.astype(jnp.int32)

    cnt = lax.fori_loop(0, MAX_DETECTIONS, step, jnp.int32(0))
    out_v_ref[...] = cnt.reshape(1, 1)


@jax.jit
def kernel(boxes, scores):
    b0 = boxes[0]
    s0 = scores[0]
    bpad = jnp.pad(b0, ((0, _NPAD - _N), (0, 0)))
    boxes3 = bpad.reshape(_NPAD, 1, 4)
    coords = bpad.T.reshape(4, _R, _L)
    spad = jnp.pad(s0.T, ((0, 0), (0, _NPAD - _N)))
    scores_t = spad.reshape(_C, _R, _L)

    out_shapes = (
        jax.ShapeDtypeStruct((MAX_DETECTIONS, 1, 4), jnp.float32),
        jax.ShapeDtypeStruct((MAX_DETECTIONS, 1, 1), jnp.float32),
        jax.ShapeDtypeStruct((MAX_DETECTIONS, 1, 1), jnp.int32),
        jax.ShapeDtypeStruct((1, 1), jnp.int32),
    )
    ob, osc, ol, vd = pl.pallas_call(
        _nms_body,
        out_shape=out_shapes,
        scratch_shapes=[
            pltpu.VMEM((_C, _R, _L), jnp.float32),
            pltpu.VMEM((1, _L), jnp.float32),
            pltpu.VMEM((1, _L), jnp.int32),
            pltpu.VMEM((_R, _L), jnp.int32),
            pltpu.VMEM((_R, _L), jnp.float32),
        ],
    )(scores_t, boxes3, coords)
    return (ob.reshape(1, MAX_DETECTIONS, 4),
            osc.reshape(1, MAX_DETECTIONS),
            ol.reshape(1, MAX_DETECTIONS),
            vd.reshape(1))


# speedup vs baseline: 42.1554x; 1.5000x over previous
import functools

import jax
import jax.numpy as jnp
from jax import lax
from jax.experimental import pallas as pl
from jax.experimental.pallas import tpu as pltpu
from jax.experimental.pallas import tpu_sc as plsc

SCORE_THRESHOLD = 0.05
IOU_THRESHOLD = 0.5
MAX_DETECTIONS = 100
NEG = -1e30

_N = 5000
_C = 80
_NPAD = 5120
_SEG = 400
_NSEG = 1024
_TOT = _SEG * _NSEG
_W = 32
_CHUNK = _TOT // _W
_SEG_PER_W = _NSEG // _W
_BIG = 2**30

@functools.lru_cache(maxsize=None)
def _get_mesh():
    return plsc.VectorSubcoreMesh(core_axis_name="c", subcore_axis_name="s",
                                  num_cores=2, num_subcores=16)


def _wid():
    return lax.axis_index("s") * 2 + lax.axis_index("c")


def _get1f(ref, i):
    lane = lax.iota(jnp.int32, 16)
    base = (i // 16) * 16
    blk = ref[pl.ds(base, 16)]
    return jnp.max(jnp.where(lane == i - base, blk, NEG))


def _set1(ref, i, val):
    lane = lax.iota(jnp.int32, 16)
    base = (i // 16) * 16
    blk = ref[pl.ds(base, 16)]
    ref[pl.ds(base, 16)] = jnp.where(lane == i - base, val, blk)


def _segmax_body(scores_hbm, segmax_hbm, chunk_v, smax_v):
    w = _wid()
    pltpu.sync_copy(scores_hbm.at[pl.ds(w * _CHUNK, _CHUNK)], chunk_v)

    def per_seg(s_local, _):
        base = s_local * _SEG

        def red(j, m):
            return jnp.maximum(m, chunk_v[pl.ds(base + j * 16, 16)])

        m = lax.fori_loop(0, _SEG // 16, red, jnp.full((16,), NEG, jnp.float32))
        _set1(smax_v, s_local, jnp.max(m))
        return 0

    lax.fori_loop(0, _SEG_PER_W, per_seg, 0)
    pltpu.sync_copy(smax_v, segmax_hbm.at[pl.ds(w * _SEG_PER_W, _SEG_PER_W)])


def _consumer_body(scores_hbm, segmax_hbm, boxes4_hbm,
                   out_b_hbm, out_s_hbm, out_l_hbm, out_v_hbm,
                   spm, sm_v, sum_v, seg_v, blk_v,
                   by1_v, bx1_v, by2_v, bx2_v,
                   ay1_v, ax1_v, ay2_v, ax2_v, aar_v, ac_v,
                   ob_v, os_v, ol_v, ov_v):
    w = _wid()

    @pl.when(w == 0)
    def _():
        lane = lax.iota(jnp.int32, 16)

        pltpu.sync_copy(scores_hbm, spm)
        pltpu.sync_copy(segmax_hbm, sm_v)
        pltpu.sync_copy(boxes4_hbm.at[0], by1_v)
        pltpu.sync_copy(boxes4_hbm.at[1], bx1_v)
        pltpu.sync_copy(boxes4_hbm.at[2], by2_v)
        pltpu.sync_copy(boxes4_hbm.at[3], bx2_v)

        def init_sum(g, _):
            _set1(sum_v, g, jnp.max(sm_v[pl.ds(g * 16, 16)]))
            return 0

        lax.fori_loop(0, _NSEG // 16, init_sum, 0)

        zf = jnp.zeros((16,), jnp.float32)
        zi = jnp.zeros((16,), jnp.int32)

        def init_out(j, _):
            os_v[pl.ds(j * 16, 16)] = zf
            ol_v[pl.ds(j * 16, 16)] = zi
            ay1_v[pl.ds(j * 16, 16)] = zf
            ax1_v[pl.ds(j * 16, 16)] = zf
            ay2_v[pl.ds(j * 16, 16)] = zf
            ax2_v[pl.ds(j * 16, 16)] = zf
            aar_v[pl.ds(j * 16, 16)] = zf
            ac_v[pl.ds(j * 16, 16)] = jnp.full((16,), -1, jnp.int32)
            for k in range(4):
                ob_v[k, pl.ds(j * 16, 16)] = zf
            return 0

        lax.fori_loop(0, 8, init_out, 0)

        def best():
            def mred(g, mv):
                return jnp.maximum(mv, sum_v[pl.ds(g * 16, 16)])

            mv = lax.fori_loop(0, 4, mred, jnp.full((16,), NEG, jnp.float32))
            m = jnp.max(mv)

            def gfind(g, gm):
                blk = sum_v[pl.ds(g * 16, 16)]
                cand = jnp.where(blk == m, g * 16 + lane, _BIG)
                return jnp.minimum(gm, jnp.min(cand))

            g_star = lax.fori_loop(0, 4, gfind, jnp.int32(_BIG))
            blk = sm_v[pl.ds(g_star * 16, 16)]
            seg = jnp.min(jnp.where(blk == m, g_star * 16 + lane, _BIG))
            return m, seg

        m0, seg0 = best()

        def cond(carry):
            cnt, m, _ = carry
            return (cnt < MAX_DETECTIONS) & (m > SCORE_THRESHOLD)

        def body(carry):
            cnt, m, seg = carry
            segbase = seg * _SEG
            pltpu.sync_copy(spm.at[pl.ds(segbase, _SEG)], seg_v)

            def pfind(j, pm):
                v = seg_v[pl.ds(j * 16, 16)]
                cand = jnp.where(v == m, j * 16 + lane, _BIG)
                return jnp.minimum(pm, jnp.min(cand))

            p = lax.fori_loop(0, _SEG // 16, pfind, jnp.int32(_BIG))
            flat = segbase + p
            b = flat // _C
            c = flat % _C

            bj = p // 16
            v = seg_v[pl.ds(bj * 16, 16)]
            v = jnp.where(lane == (p - bj * 16), NEG, v)
            seg_v[pl.ds(bj * 16, 16)] = v
            blk_v[...] = v
            pltpu.sync_copy(blk_v, spm.at[pl.ds(segbase + bj * 16, 16)])

            def sred(j, mv):
                return jnp.maximum(mv, seg_v[pl.ds(j * 16, 16)])

            msv = lax.fori_loop(0, _SEG // 16, sred,
                                jnp.full((16,), NEG, jnp.float32))
            _set1(sm_v, seg, jnp.max(msv))
            g = seg // 16
            _set1(sum_v, g, jnp.max(sm_v[pl.ds(g * 16, 16)]))

            y1 = _get1f(by1_v, b)
            x1 = _get1f(bx1_v, b)
            y2 = _get1f(by2_v, b)
            x2 = _get1f(bx2_v, b)
            a2 = (y2 - y1) * (x2 - x1)

            def chk(k, nv):
                qy1 = jnp.maximum(y1, ay1_v[pl.ds(k * 16, 16)])
                qx1 = jnp.maximum(x1, ax1_v[pl.ds(k * 16, 16)])
                qy2 = jnp.minimum(y2, ay2_v[pl.ds(k * 16, 16)])
                qx2 = jnp.minimum(x2, ax2_v[pl.ds(k * 16, 16)])
                inter = (jnp.maximum(qy2 - qy1, 0.0)
                         * jnp.maximum(qx2 - qx1, 0.0))
                denom = jnp.maximum(aar_v[pl.ds(k * 16, 16)] + a2 - inter,
                                    1e-9)
                iou = inter / denom
                hit = (iou > IOU_THRESHOLD) & (ac_v[pl.ds(k * 16, 16)] == c)
                return nv + jnp.sum(hit.astype(jnp.int32))

            nviol = lax.fori_loop(0, 7, chk, jnp.int32(0))
            accept = nviol == 0
            slot = jnp.where(accept, cnt, 127)

            _set1(ay1_v, slot, y1)
            _set1(ax1_v, slot, x1)
            _set1(ay2_v, slot, y2)
            _set1(ax2_v, slot, x2)
            _set1(aar_v, slot, a2)
            _set1(ac_v, slot, c)
            _set1(os_v, slot, m)
            _set1(ol_v, slot, c)
            lane16 = lax.iota(jnp.int32, 16)
            sbase = (slot // 16) * 16
            soff = slot - sbase
            for k, cv in enumerate((y1, x1, y2, x2)):
                blk = ob_v[k, pl.ds(sbase, 16)]
                ob_v[k, pl.ds(sbase, 16)] = jnp.where(lane16 == soff, cv, blk)

            m2, seg2 = best()
            return cnt + accept.astype(jnp.int32), m2, seg2

        cnt, _, _ = lax.while_loop(cond, body, (jnp.int32(0), m0, seg0))

        ov_v[...] = jnp.where(lane == 0, cnt, 0)
        for k in range(4):
            pltpu.sync_copy(ob_v.at[k], out_b_hbm.at[k])
        pltpu.sync_copy(os_v, out_s_hbm)
        pltpu.sync_copy(ol_v, out_l_hbm)
        pltpu.sync_copy(ov_v, out_v_hbm)


@jax.jit
def kernel(boxes, scores):
    sflat = jnp.pad(scores[0].reshape(-1), (0, _TOT - _N * _C))
    boxes4 = jnp.pad(boxes[0], ((0, _NPAD - _N), (0, 0))).T

    k1 = pl.kernel(
        _segmax_body,
        out_type=jax.ShapeDtypeStruct((_NSEG,), jnp.float32),
        mesh=_get_mesh(),
        scratch_types=[
            pltpu.VMEM((_CHUNK,), jnp.float32),
            pltpu.VMEM((_SEG_PER_W,), jnp.float32),
        ],
        compiler_params=pltpu.CompilerParams(needs_layout_passes=False),
    )
    segmax = k1(sflat)

    k2 = pl.kernel(
        _consumer_body,
        out_type=(
            jax.ShapeDtypeStruct((4, 128), jnp.float32),
            jax.ShapeDtypeStruct((128,), jnp.float32),
            jax.ShapeDtypeStruct((128,), jnp.int32),
            jax.ShapeDtypeStruct((16,), jnp.int32),
        ),
        mesh=_get_mesh(),
        scratch_types=[
            pltpu.VMEM_SHARED((_TOT,), jnp.float32),
            pltpu.VMEM((_NSEG,), jnp.float32),
            pltpu.VMEM((_NSEG // 16,), jnp.float32),
            pltpu.VMEM((_SEG,), jnp.float32),
            pltpu.VMEM((16,), jnp.float32),
            pltpu.VMEM((_NPAD,), jnp.float32),
            pltpu.VMEM((_NPAD,), jnp.float32),
            pltpu.VMEM((_NPAD,), jnp.float32),
            pltpu.VMEM((_NPAD,), jnp.float32),
            pltpu.VMEM((128,), jnp.float32),
            pltpu.VMEM((128,), jnp.float32),
            pltpu.VMEM((128,), jnp.float32),
            pltpu.VMEM((128,), jnp.float32),
            pltpu.VMEM((128,), jnp.float32),
            pltpu.VMEM((128,), jnp.int32),
            pltpu.VMEM((4, 128), jnp.float32),
            pltpu.VMEM((128,), jnp.float32),
            pltpu.VMEM((128,), jnp.int32),
            pltpu.VMEM((16,), jnp.int32),
        ],
        compiler_params=pltpu.CompilerParams(needs_layout_passes=False),
    )
    ob4, osc, ol, ov = k2(sflat, segmax, boxes4)

    return (ob4.T[:MAX_DETECTIONS].reshape(1, MAX_DETECTIONS, 4),
            osc[:MAX_DETECTIONS].reshape(1, MAX_DETECTIONS),
            ol[:MAX_DETECTIONS].reshape(1, MAX_DETECTIONS),
            ov[0].reshape(1))


# speedup vs baseline: 43.8436x; 1.0400x over previous
import functools

import jax
import jax.numpy as jnp
from jax import lax
from jax.experimental import pallas as pl
from jax.experimental.pallas import tpu as pltpu
from jax.experimental.pallas import tpu_sc as plsc

SCORE_THRESHOLD = 0.05
IOU_THRESHOLD = 0.5
MAX_DETECTIONS = 100
NEG = -1e30

_N = 5000
_C = 80
_NPAD = 5120
_SEG = 400
_NSEG = 1024
_TOT = _SEG * _NSEG
_W = 32
_CHUNK = _TOT // _W
_SEG_PER_W = _NSEG // _W
_BIG = 2**30

@functools.lru_cache(maxsize=None)
def _get_mesh():
    return plsc.VectorSubcoreMesh(core_axis_name="c", subcore_axis_name="s",
                                  num_cores=2, num_subcores=16)


def _wid():
    return lax.axis_index("s") * 2 + lax.axis_index("c")


def _get1f(ref, i):
    lane = lax.iota(jnp.int32, 16)
    base = (i // 16) * 16
    blk = ref[pl.ds(base, 16)]
    return jnp.max(jnp.where(lane == i - base, blk, NEG))


def _set1(ref, i, val):
    lane = lax.iota(jnp.int32, 16)
    base = (i // 16) * 16
    blk = ref[pl.ds(base, 16)]
    ref[pl.ds(base, 16)] = jnp.where(lane == i - base, val, blk)


def _segmax_body(scores_hbm, segmax_hbm, chunk_v, smax_v):
    w = _wid()
    pltpu.sync_copy(scores_hbm.at[pl.ds(w * _CHUNK, _CHUNK)], chunk_v)

    def per_seg(s_local, _):
        base = s_local * _SEG

        def red(j, m):
            return jnp.maximum(m, chunk_v[pl.ds(base + j * 16, 16)])

        m = lax.fori_loop(0, _SEG // 16, red, jnp.full((16,), NEG, jnp.float32))
        _set1(smax_v, s_local, jnp.max(m))
        return 0

    lax.fori_loop(0, _SEG_PER_W, per_seg, 0)
    pltpu.sync_copy(smax_v, segmax_hbm.at[pl.ds(w * _SEG_PER_W, _SEG_PER_W)])


def _consumer_body(scores_hbm, segmax_hbm, boxes4_hbm,
                   out_b_hbm, out_s_hbm, out_l_hbm, out_v_hbm,
                   spm, sm_v, sum_v, seg_v, blk_v, bco_v,
                   ay1_v, ax1_v, ay2_v, ax2_v, aar_v, ac_v,
                   ob_v, os_v, ol_v, ov_v):
    w = _wid()

    @pl.when(w == 0)
    def _():
        lane = lax.iota(jnp.int32, 16)

        pltpu.sync_copy(scores_hbm, spm)
        pltpu.sync_copy(segmax_hbm, sm_v)
        pltpu.sync_copy(boxes4_hbm, bco_v)

        def init_sum(g, _):
            _set1(sum_v, g, jnp.max(sm_v[pl.ds(g * 16, 16)]))
            return 0

        lax.fori_loop(0, _NSEG // 16, init_sum, 0)

        zf = jnp.zeros((16,), jnp.float32)
        zi = jnp.zeros((16,), jnp.int32)

        def init_out(j, _):
            os_v[pl.ds(j * 16, 16)] = zf
            ol_v[pl.ds(j * 16, 16)] = zi
            ay1_v[pl.ds(j * 16, 16)] = zf
            ax1_v[pl.ds(j * 16, 16)] = zf
            ay2_v[pl.ds(j * 16, 16)] = zf
            ax2_v[pl.ds(j * 16, 16)] = zf
            aar_v[pl.ds(j * 16, 16)] = zf
            ac_v[pl.ds(j * 16, 16)] = jnp.full((16,), -1, jnp.int32)
            for k in range(4):
                ob_v[k, pl.ds(j * 16, 16)] = zf
            return 0

        lax.fori_loop(0, 8, init_out, 0)

        def best():
            def mred(g, mv):
                return jnp.maximum(mv, sum_v[pl.ds(g * 16, 16)])

            mv = lax.fori_loop(0, 4, mred, jnp.full((16,), NEG, jnp.float32))
            m = jnp.max(mv)

            def gfind(g, gm):
                blk = sum_v[pl.ds(g * 16, 16)]
                return jnp.minimum(gm, jnp.where(blk == m, g * 16 + lane, _BIG))

            g_star = jnp.min(
                lax.fori_loop(0, 4, gfind, jnp.full((16,), _BIG, jnp.int32)))
            blk = sm_v[pl.ds(g_star * 16, 16)]
            seg = jnp.min(jnp.where(blk == m, g_star * 16 + lane, _BIG))
            return m, seg

        m0, seg0 = best()

        def cond(carry):
            cnt, m, _ = carry
            return (cnt < MAX_DETECTIONS) & (m > SCORE_THRESHOLD)

        def body(carry):
            cnt, m, seg = carry
            segbase = seg * _SEG
            pltpu.sync_copy(spm.at[pl.ds(segbase, _SEG)], seg_v)

            def pfind(j, pm):
                v = seg_v[pl.ds(j * 16, 16)]
                return jnp.minimum(pm, jnp.where(v == m, j * 16 + lane, _BIG))

            p = jnp.min(lax.fori_loop(0, _SEG // 16, pfind,
                                      jnp.full((16,), _BIG, jnp.int32)))
            flat = segbase + p
            b = flat // _C
            c = flat % _C

            bj = p // 16
            v = seg_v[pl.ds(bj * 16, 16)]
            v = jnp.where(lane == (p - bj * 16), NEG, v)
            seg_v[pl.ds(bj * 16, 16)] = v
            blk_v[...] = v
            pltpu.sync_copy(blk_v, spm.at[pl.ds(segbase + bj * 16, 16)])

            def sred(j, mv):
                return jnp.maximum(mv, seg_v[pl.ds(j * 16, 16)])

            msv = lax.fori_loop(0, _SEG // 16, sred,
                                jnp.full((16,), NEG, jnp.float32))
            _set1(sm_v, seg, jnp.max(msv))
            g = seg // 16
            _set1(sum_v, g, jnp.max(sm_v[pl.ds(g * 16, 16)]))

            gidx = jnp.where(lane < 4, b + _NPAD * lane, b)
            bco = plsc.load_gather(bco_v, [gidx])
            y1 = bco[0]
            x1 = bco[1]
            y2 = bco[2]
            x2 = bco[3]
            a2 = (y2 - y1) * (x2 - x1)

            def chk(k, nv):
                qy1 = jnp.maximum(y1, ay1_v[pl.ds(k * 16, 16)])
                qx1 = jnp.maximum(x1, ax1_v[pl.ds(k * 16, 16)])
                qy2 = jnp.minimum(y2, ay2_v[pl.ds(k * 16, 16)])
                qx2 = jnp.minimum(x2, ax2_v[pl.ds(k * 16, 16)])
                inter = (jnp.maximum(qy2 - qy1, 0.0)
                         * jnp.maximum(qx2 - qx1, 0.0))
                denom = jnp.maximum(aar_v[pl.ds(k * 16, 16)] + a2 - inter,
                                    1e-9)
                iou = inter / denom
                hit = (iou > IOU_THRESHOLD) & (ac_v[pl.ds(k * 16, 16)] == c)
                return nv + hit.astype(jnp.int32)

            nviol = lax.fori_loop(0, 7, chk, jnp.zeros((16,), jnp.int32))
            accept = jnp.sum(nviol) == 0
            slot = jnp.where(accept, cnt, 127)

            _set1(ay1_v, slot, y1)
            _set1(ax1_v, slot, x1)
            _set1(ay2_v, slot, y2)
            _set1(ax2_v, slot, x2)
            _set1(aar_v, slot, a2)
            _set1(ac_v, slot, c)
            _set1(os_v, slot, m)
            _set1(ol_v, slot, c)
            lane16 = lax.iota(jnp.int32, 16)
            sbase = (slot // 16) * 16
            soff = slot - sbase
            for k, cv in enumerate((y1, x1, y2, x2)):
                blk = ob_v[k, pl.ds(sbase, 16)]
                ob_v[k, pl.ds(sbase, 16)] = jnp.where(lane16 == soff, cv, blk)

            m2, seg2 = best()
            return cnt + accept.astype(jnp.int32), m2, seg2

        cnt, _, _ = lax.while_loop(cond, body, (jnp.int32(0), m0, seg0))

        ov_v[...] = jnp.where(lane == 0, cnt, 0)
        for k in range(4):
            pltpu.sync_copy(ob_v.at[k], out_b_hbm.at[k])
        pltpu.sync_copy(os_v, out_s_hbm)
        pltpu.sync_copy(ol_v, out_l_hbm)
        pltpu.sync_copy(ov_v, out_v_hbm)


@jax.jit
def kernel(boxes, scores):
    sflat = jnp.pad(scores[0].reshape(-1), (0, _TOT - _N * _C))
    boxes4 = jnp.pad(boxes[0], ((0, _NPAD - _N), (0, 0))).T.reshape(-1)

    k1 = pl.kernel(
        _segmax_body,
        out_type=jax.ShapeDtypeStruct((_NSEG,), jnp.float32),
        mesh=_get_mesh(),
        scratch_types=[
            pltpu.VMEM((_CHUNK,), jnp.float32),
            pltpu.VMEM((_SEG_PER_W,), jnp.float32),
        ],
        compiler_params=pltpu.CompilerParams(needs_layout_passes=False),
    )
    segmax = k1(sflat)

    k2 = pl.kernel(
        _consumer_body,
        out_type=(
            jax.ShapeDtypeStruct((4, 128), jnp.float32),
            jax.ShapeDtypeStruct((128,), jnp.float32),
            jax.ShapeDtypeStruct((128,), jnp.int32),
            jax.ShapeDtypeStruct((16,), jnp.int32),
        ),
        mesh=_get_mesh(),
        scratch_types=[
            pltpu.VMEM_SHARED((_TOT,), jnp.float32),
            pltpu.VMEM((_NSEG,), jnp.float32),
            pltpu.VMEM((_NSEG // 16,), jnp.float32),
            pltpu.VMEM((_SEG,), jnp.float32),
            pltpu.VMEM((16,), jnp.float32),
            pltpu.VMEM((4 * _NPAD,), jnp.float32),
            pltpu.VMEM((128,), jnp.float32),
            pltpu.VMEM((128,), jnp.float32),
            pltpu.VMEM((128,), jnp.float32),
            pltpu.VMEM((128,), jnp.float32),
            pltpu.VMEM((128,), jnp.float32),
            pltpu.VMEM((128,), jnp.int32),
            pltpu.VMEM((4, 128), jnp.float32),
            pltpu.VMEM((128,), jnp.float32),
            pltpu.VMEM((128,), jnp.int32),
            pltpu.VMEM((16,), jnp.int32),
        ],
        compiler_params=pltpu.CompilerParams(needs_layout_passes=False),
    )
    ob4, osc, ol, ov = k2(sflat, segmax, boxes4)

    return (ob4.T[:MAX_DETECTIONS].reshape(1, MAX_DETECTIONS, 4),
            osc[:MAX_DETECTIONS].reshape(1, MAX_DETECTIONS),
            ol[:MAX_DETECTIONS].reshape(1, MAX_DETECTIONS),
            ov[0].reshape(1))


# speedup vs baseline: 46.2681x; 1.0553x over previous
import functools

import jax
import jax.numpy as jnp
from jax import lax
from jax.experimental import pallas as pl
from jax.experimental.pallas import tpu as pltpu
from jax.experimental.pallas import tpu_sc as plsc

SCORE_THRESHOLD = 0.05
IOU_THRESHOLD = 0.5
MAX_DETECTIONS = 100
NEG = -1e30

_N = 5000
_C = 80
_NPAD = 5120
_SEG = 256
_W = 16
_SEG_PER_W = 100
_SLOT_PER_W = 112
_NSEG = _W * _SLOT_PER_W
_CHUNK = _SEG_PER_W * _SEG
_TOT = _W * _CHUNK
_NBLK = _NSEG * 16
_BIG = 2**30


@functools.lru_cache(maxsize=None)
def _get_mesh():
    return plsc.VectorSubcoreMesh(core_axis_name="c", subcore_axis_name="s",
                                  num_cores=2, num_subcores=16)


def _set1(ref, i, val):
    lane = lax.iota(jnp.int32, 16)
    base = (i // 16) * 16
    blk = ref[pl.ds(base, 16)]
    ref[pl.ds(base, 16)] = jnp.where(lane == i - base, val, blk)


def _nms_body(scores_hbm, boxes4_hbm,
              out_b_hbm, out_s_hbm, out_l_hbm, out_v_hbm,
              spm_sc, spm_bm, spm_sm,
              chunk_v, bm_v, smax_v,
              sm_v, l2_v, bmg_v, blk_v, bco_v,
              ay1_v, ax1_v, ay2_v, ax2_v, aar_v, ac_v,
              ob_v, os_v, ol_v, ov_v):
    cid = lax.axis_index("c")
    sid = lax.axis_index("s")
    lane = lax.iota(jnp.int32, 16)

    @pl.when(cid == 0)
    def _phase1():
        base = sid * _CHUNK
        pltpu.sync_copy(scores_hbm.at[pl.ds(base, _CHUNK)], chunk_v)
        pltpu.sync_copy(chunk_v, spm_sc.at[pl.ds(base, _CHUNK)])

        def init_pad(j, _):
            smax_v[pl.ds(j * 16, 16)] = jnp.full((16,), NEG, jnp.float32)
            return 0

        lax.fori_loop(0, _SLOT_PER_W // 16, init_pad, 0)

        def per_seg(sl, _):
            sbase = sl * _SEG

            def bred(k, mv):
                idx = sbase + lane * 16 + k
                return jnp.maximum(mv, plsc.load_gather(chunk_v, [idx]))

            mv = lax.fori_loop(0, 16, bred, jnp.full((16,), NEG, jnp.float32))
            bm_v[pl.ds(sl * 16, 16)] = mv
            _set1(smax_v, sl, jnp.max(mv))
            return 0

        lax.fori_loop(0, _SEG_PER_W, per_seg, 0)
        pltpu.sync_copy(bm_v, spm_bm.at[pl.ds(sid * _SLOT_PER_W * 16,
                                              _SEG_PER_W * 16)])
        pltpu.sync_copy(smax_v, spm_sm.at[pl.ds(sid * _SLOT_PER_W,
                                                _SLOT_PER_W)])

    plsc.subcore_barrier()

    @pl.when((cid == 0) & (sid == 0))
    def _phase2():
        pltpu.sync_copy(spm_sm, sm_v)
        pltpu.sync_copy(spm_bm, bmg_v)
        pltpu.sync_copy(boxes4_hbm, bco_v)

        def init_l2(gg, _):
            def gred(k, mv):
                idx = gg * 256 + lane * 16 + k
                return jnp.maximum(mv, plsc.load_gather(sm_v, [idx]))

            mv = lax.fori_loop(0, 16, gred, jnp.full((16,), NEG, jnp.float32))
            l2_v[pl.ds(gg * 16, 16)] = mv
            return 0

        lax.fori_loop(0, _NSEG // 256, init_l2, 0)

        zf = jnp.zeros((16,), jnp.float32)
        zi = jnp.zeros((16,), jnp.int32)

        def init_out(j, _):
            os_v[pl.ds(j * 16, 16)] = zf
            ol_v[pl.ds(j * 16, 16)] = zi
            ay1_v[pl.ds(j * 16, 16)] = zf
            ax1_v[pl.ds(j * 16, 16)] = zf
            ay2_v[pl.ds(j * 16, 16)] = zf
            ax2_v[pl.ds(j * 16, 16)] = zf
            aar_v[pl.ds(j * 16, 16)] = zf
            ac_v[pl.ds(j * 16, 16)] = jnp.full((16,), -1, jnp.int32)
            for k in range(4):
                ob_v[k, pl.ds(j * 16, 16)] = zf
            return 0

        lax.fori_loop(0, 8, init_out, 0)

        def best():
            def mred(g, mv):
                return jnp.maximum(mv, l2_v[pl.ds(g * 16, 16)])

            mv = lax.fori_loop(0, 7, mred, jnp.full((16,), NEG, jnp.float32))
            m = jnp.max(mv)

            def gfind(g, gm):
                blk = l2_v[pl.ds(g * 16, 16)]
                return jnp.minimum(gm, jnp.where(blk == m, g * 16 + lane,
                                                 _BIG))

            g_star = jnp.min(
                lax.fori_loop(0, 7, gfind, jnp.full((16,), _BIG, jnp.int32)))
            blk = sm_v[pl.ds(g_star * 16, 16)]
            seg = jnp.min(jnp.where(blk == m, g_star * 16 + lane, _BIG))
            return m, seg

        m0, seg0 = best()

        def cond(carry):
            cnt, m, _ = carry
            return (cnt < MAX_DETECTIONS) & (m > SCORE_THRESHOLD)

        def body(carry):
            cnt, m, seg = carry

            bmv = bmg_v[pl.ds(seg * 16, 16)]
            bj = jnp.min(jnp.where(bmv == m, lane, _BIG))
            ebase = ((seg // _SLOT_PER_W) * _CHUNK
                     + (seg % _SLOT_PER_W) * _SEG + bj * 16)
            pltpu.sync_copy(spm_sc.at[pl.ds(ebase, 16)], blk_v)
            v = blk_v[...]
            off = jnp.min(jnp.where(v == m, lane, _BIG))
            flat = ebase + off
            b = flat // _C
            c = flat % _C

            v = jnp.where(lane == off, NEG, v)
            blk_v[...] = v
            pltpu.sync_copy(blk_v, spm_sc.at[pl.ds(ebase, 16)])
            bmv = jnp.where(lane == bj, jnp.max(v), bmv)
            bmg_v[pl.ds(seg * 16, 16)] = bmv
            _set1(sm_v, seg, jnp.max(bmv))
            g = seg // 16
            _set1(l2_v, g, jnp.max(sm_v[pl.ds(g * 16, 16)]))

            gidx = jnp.where(lane < 4, b + _NPAD * lane, b)
            bco = plsc.load_gather(bco_v, [gidx])
            y1 = bco[0]
            x1 = bco[1]
            y2 = bco[2]
            x2 = bco[3]
            a2 = (y2 - y1) * (x2 - x1)

            def chk(k, nv):
                qy1 = jnp.maximum(y1, ay1_v[pl.ds(k * 16, 16)])
                qx1 = jnp.maximum(x1, ax1_v[pl.ds(k * 16, 16)])
                qy2 = jnp.minimum(y2, ay2_v[pl.ds(k * 16, 16)])
                qx2 = jnp.minimum(x2, ax2_v[pl.ds(k * 16, 16)])
                inter = (jnp.maximum(qy2 - qy1, 0.0)
                         * jnp.maximum(qx2 - qx1, 0.0))
                denom = jnp.maximum(aar_v[pl.ds(k * 16, 16)] + a2 - inter,
                                    1e-9)
                iou = inter / denom
                hit = (iou > IOU_THRESHOLD) & (ac_v[pl.ds(k * 16, 16)] == c)
                return nv + hit.astype(jnp.int32)

            nviol = lax.fori_loop(0, (cnt + 15) // 16, chk,
                                  jnp.zeros((16,), jnp.int32))
            accept = jnp.sum(nviol) == 0
            slot = jnp.where(accept, cnt, 127)

            _set1(ay1_v, slot, y1)
            _set1(ax1_v, slot, x1)
            _set1(ay2_v, slot, y2)
            _set1(ax2_v, slot, x2)
            _set1(aar_v, slot, a2)
            _set1(ac_v, slot, c)
            _set1(os_v, slot, m)
            _set1(ol_v, slot, c)
            sbase = (slot // 16) * 16
            soff = slot - sbase
            for k, cv in enumerate((y1, x1, y2, x2)):
                blk = ob_v[k, pl.ds(sbase, 16)]
                ob_v[k, pl.ds(sbase, 16)] = jnp.where(lane == soff, cv, blk)

            m2, seg2 = best()
            return cnt + accept.astype(jnp.int32), m2, seg2

        cnt, _, _ = lax.while_loop(cond, body, (jnp.int32(0), m0, seg0))

        ov_v[...] = jnp.where(lane == 0, cnt, 0)
        for k in range(4):
            pltpu.sync_copy(ob_v.at[k], out_b_hbm.at[k])
        pltpu.sync_copy(os_v, out_s_hbm)
        pltpu.sync_copy(ol_v, out_l_hbm)
        pltpu.sync_copy(ov_v, out_v_hbm)


@jax.jit
def kernel(boxes, scores):
    sflat = jnp.pad(scores[0].reshape(-1), (0, _TOT - _N * _C))
    boxes4 = jnp.pad(boxes[0], ((0, _NPAD - _N), (0, 0))).T.reshape(-1)

    k = pl.kernel(
        _nms_body,
        out_type=(
            jax.ShapeDtypeStruct((4, 128), jnp.float32),
            jax.ShapeDtypeStruct((128,), jnp.float32),
            jax.ShapeDtypeStruct((128,), jnp.int32),
            jax.ShapeDtypeStruct((16,), jnp.int32),
        ),
        mesh=_get_mesh(),
        scratch_types=[
            pltpu.VMEM_SHARED((_TOT,), jnp.float32),
            pltpu.VMEM_SHARED((_NBLK,), jnp.float32),
            pltpu.VMEM_SHARED((_NSEG,), jnp.float32),
            pltpu.VMEM((_CHUNK,), jnp.float32),
            pltpu.VMEM((_SEG_PER_W * 16,), jnp.float32),
            pltpu.VMEM((_SLOT_PER_W,), jnp.float32),
            pltpu.VMEM((_NSEG,), jnp.float32),
            pltpu.VMEM((_NSEG // 16,), jnp.float32),
            pltpu.VMEM((_NBLK,), jnp.float32),
            pltpu.VMEM((16,), jnp.float32),
            pltpu.VMEM((4 * _NPAD,), jnp.float32),
            pltpu.VMEM((128,), jnp.float32),
            pltpu.VMEM((128,), jnp.float32),
            pltpu.VMEM((128,), jnp.float32),
            pltpu.VMEM((128,), jnp.float32),
            pltpu.VMEM((128,), jnp.float32),
            pltpu.VMEM((128,), jnp.int32),
            pltpu.VMEM((4, 128), jnp.float32),
            pltpu.VMEM((128,), jnp.float32),
            pltpu.VMEM((128,), jnp.int32),
            pltpu.VMEM((16,), jnp.int32),
        ],
        compiler_params=pltpu.CompilerParams(needs_layout_passes=False),
    )
    ob4, osc, ol, ov = k(sflat, boxes4)

    return (ob4.T[:MAX_DETECTIONS].reshape(1, MAX_DETECTIONS, 4),
            osc[:MAX_DETECTIONS].reshape(1, MAX_DETECTIONS),
            ol[:MAX_DETECTIONS].reshape(1, MAX_DETECTIONS),
            ov[0].reshape(1))


# speedup vs baseline: 49.1783x; 1.0629x over previous
import functools

import jax
import jax.numpy as jnp
from jax import lax
from jax.experimental import pallas as pl
from jax.experimental.pallas import tpu as pltpu
from jax.experimental.pallas import tpu_sc as plsc

SCORE_THRESHOLD = 0.05
IOU_THRESHOLD = 0.5
MAX_DETECTIONS = 100
NEG = -1e30

_N = 5000
_C = 80
_NPAD = 5120
_SEG = 256
_W = 16
_SEG_PER_W = 100
_SLOT_PER_W = 112
_NSEG = _W * _SLOT_PER_W
_CHUNK = _SEG_PER_W * _SEG
_TOT = _W * _CHUNK
_BIG = 2**30
_UNK = -1


@functools.lru_cache(maxsize=None)
def _get_mesh():
    return plsc.VectorSubcoreMesh(core_axis_name="c", subcore_axis_name="s",
                                  num_cores=2, num_subcores=16)


def _set1(ref, i, val):
    lane = lax.iota(jnp.int32, 16)
    base = (i // 16) * 16
    blk = ref[pl.ds(base, 16)]
    ref[pl.ds(base, 16)] = jnp.where(lane == i - base, val, blk)


def _get1(ref, i):
    idx = jnp.full((16,), 0, jnp.int32) + i
    return plsc.load_gather(ref, [idx])[0]


def _nms_body(scores_hbm, boxes4_hbm,
              out_b_hbm, out_s_hbm, out_l_hbm, out_v_hbm,
              spm_sc, spm_sm, spm_p1, spm_nx,
              chunk_v, smax_v, p1st_v, nxst_v,
              sm_v, l2_v, cp_v, lv_v, lp_v, nx_v, seg_v, bco_v,
              ay1_v, ax1_v, ay2_v, ax2_v, aar_v, ac_v,
              ob_v, os_v, ol_v, ov_v):
    cid = lax.axis_index("c")
    sid = lax.axis_index("s")
    lane = lax.iota(jnp.int32, 16)

    @pl.when(cid == 0)
    def _phase1():
        base = sid * _CHUNK
        pltpu.sync_copy(scores_hbm.at[pl.ds(base, _CHUNK)], chunk_v)
        pltpu.sync_copy(chunk_v, spm_sc.at[pl.ds(base, _CHUNK)])

        def per_group(sg, _):
            slot = sg * 16 + lane
            real = slot < _SEG_PER_W
            lbase = jnp.where(real, slot * _SEG, 0)
            fbase = base + lbase

            def step(k, carry):
                mv, pm, sv = carry
                g = plsc.load_gather(chunk_v, [lbase + k])
                gt = g > mv
                sv = jnp.where(gt, mv, jnp.maximum(sv, g))
                pm = jnp.where(gt, fbase + k, pm)
                mv = jnp.maximum(mv, g)
                return mv, pm, sv

            mv, pm, sv = lax.fori_loop(
                0, _SEG, step,
                (jnp.full((16,), NEG, jnp.float32),
                 jnp.zeros((16,), jnp.int32),
                 jnp.full((16,), NEG, jnp.float32)))
            smax_v[pl.ds(sg * 16, 16)] = jnp.where(real, mv, NEG)
            p1st_v[pl.ds(sg * 16, 16)] = pm
            nxst_v[pl.ds(sg * 16, 16)] = jnp.where(real, sv, NEG)
            return 0

        lax.fori_loop(0, _SLOT_PER_W // 16, per_group, 0)
        off = sid * _SLOT_PER_W
        pltpu.sync_copy(smax_v, spm_sm.at[pl.ds(off, _SLOT_PER_W)])
        pltpu.sync_copy(p1st_v, spm_p1.at[pl.ds(off, _SLOT_PER_W)])
        pltpu.sync_copy(nxst_v, spm_nx.at[pl.ds(off, _SLOT_PER_W)])

    plsc.subcore_barrier()

    @pl.when((cid == 0) & (sid == 0))
    def _phase2():
        pltpu.sync_copy(spm_sm, sm_v)
        pltpu.sync_copy(spm_p1, cp_v)
        pltpu.sync_copy(spm_nx, nx_v)
        pltpu.sync_copy(boxes4_hbm, bco_v)

        def init_l2(gg, _):
            gidx = (gg * 16 + lane) * 16

            def gred(k, mvv):
                return jnp.maximum(mvv, plsc.load_gather(sm_v, [gidx + k]))

            mvv = lax.fori_loop(0, 16, gred,
                                jnp.full((16,), NEG, jnp.float32))
            l2_v[pl.ds(gg * 16, 16)] = mvv
            return 0

        lax.fori_loop(0, _NSEG // 256, init_l2, 0)

        zf = jnp.zeros((16,), jnp.float32)
        zi = jnp.zeros((16,), jnp.int32)

        def init_out(j, _):
            os_v[pl.ds(j * 16, 16)] = zf
            ol_v[pl.ds(j * 16, 16)] = zi
            ay1_v[pl.ds(j * 16, 16)] = zf
            ax1_v[pl.ds(j * 16, 16)] = zf
            ay2_v[pl.ds(j * 16, 16)] = zf
            ax2_v[pl.ds(j * 16, 16)] = zf
            aar_v[pl.ds(j * 16, 16)] = zf
            ac_v[pl.ds(j * 16, 16)] = jnp.full((16,), -1, jnp.int32)
            lv_v[pl.ds(j * 16, 16)] = zf
            lp_v[pl.ds(j * 16, 16)] = zi
            for k in range(4):
                ob_v[k, pl.ds(j * 16, 16)] = zf
            return 0

        lax.fori_loop(0, 8, init_out, 0)

        def zero_lvlp(j, _):
            lv_v[pl.ds(j * 16, 16)] = zf
            lp_v[pl.ds(j * 16, 16)] = zi
            return 0

        lax.fori_loop(8, _NSEG // 16, zero_lvlp, 0)

        def best():
            def mred(g, mvv):
                return jnp.maximum(mvv, l2_v[pl.ds(g * 16, 16)])

            mvv = lax.fori_loop(0, 7, mred, jnp.full((16,), NEG, jnp.float32))
            m = jnp.max(mvv)

            def gfind(g, gm):
                blk = l2_v[pl.ds(g * 16, 16)]
                return jnp.minimum(gm, jnp.where(blk == m, g * 16 + lane,
                                                 _BIG))

            g_star = jnp.min(
                lax.fori_loop(0, 7, gfind, jnp.full((16,), _BIG, jnp.int32)))
            blk = sm_v[pl.ds(g_star * 16, 16)]
            seg = jnp.min(jnp.where(blk == m, g_star * 16 + lane, _BIG))
            return m, seg

        m0, seg0 = best()

        def cond(carry):
            cnt, m, _ = carry
            return (cnt < MAX_DETECTIONS) & (m > SCORE_THRESHOLD)

        def body(carry):
            cnt, m, seg = carry
            cp = _get1(cp_v, seg)

            def fast(_):
                return cp, _get1(nx_v, seg)

            def slow(_):
                lv = _get1(lv_v, seg)
                lp = _get1(lp_v, seg)
                ebase = ((seg // _SLOT_PER_W) * _CHUNK
                         + (seg % _SLOT_PER_W) * _SEG)
                pltpu.sync_copy(spm_sc.at[pl.ds(ebase, _SEG)], seg_v)

                def find(j, pm):
                    v = seg_v[pl.ds(j * 16, 16)]
                    pos = ebase + j * 16 + lane
                    elig = (v < lv) | ((v == lv) & (pos > lp))
                    return jnp.minimum(pm, jnp.where(elig & (v == m), pos,
                                                     _BIG))

                p_cur = jnp.min(lax.fori_loop(
                    0, _SEG // 16, find, jnp.full((16,), _BIG, jnp.int32)))

                def nxt(j, nv):
                    v = seg_v[pl.ds(j * 16, 16)]
                    pos = ebase + j * 16 + lane
                    elig = ((v < lv) | ((v == lv) & (pos > lp))) & \
                           ((v < m) | ((v == m) & (pos > p_cur)))
                    return jnp.maximum(nv, jnp.where(elig, v, NEG))

                v_next = jnp.max(lax.fori_loop(
                    0, _SEG // 16, nxt,
                    jnp.full((16,), NEG, jnp.float32)))
                return p_cur, v_next

            p_cur, v_next = lax.cond(cp != _UNK, fast, slow, 0)

            _set1(lv_v, seg, m)
            _set1(lp_v, seg, p_cur)
            _set1(cp_v, seg, _UNK)
            _set1(sm_v, seg, v_next)
            g = seg // 16
            _set1(l2_v, g, jnp.max(sm_v[pl.ds(g * 16, 16)]))

            b = p_cur // _C
            c = p_cur % _C

            gidx = jnp.where(lane < 4, b + _NPAD * lane, b)
            bco = plsc.load_gather(bco_v, [gidx])
            y1 = bco[0]
            x1 = bco[1]
            y2 = bco[2]
            x2 = bco[3]
            a2 = (y2 - y1) * (x2 - x1)

            def chk(k, nv):
                qy1 = jnp.maximum(y1, ay1_v[pl.ds(k * 16, 16)])
                qx1 = jnp.maximum(x1, ax1_v[pl.ds(k * 16, 16)])
                qy2 = jnp.minimum(y2, ay2_v[pl.ds(k * 16, 16)])
                qx2 = jnp.minimum(x2, ax2_v[pl.ds(k * 16, 16)])
                inter = (jnp.maximum(qy2 - qy1, 0.0)
                         * jnp.maximum(qx2 - qx1, 0.0))
                denom = jnp.maximum(aar_v[pl.ds(k * 16, 16)] + a2 - inter,
                                    1e-9)
                iou = inter / denom
                hit = (iou > IOU_THRESHOLD) & (ac_v[pl.ds(k * 16, 16)] == c)
                return nv + hit.astype(jnp.int32)

            nviol = lax.fori_loop(0, (cnt + 15) // 16, chk,
                                  jnp.zeros((16,), jnp.int32))
            accept = jnp.sum(nviol) == 0
            slot = jnp.where(accept, cnt, 127)

            _set1(ay1_v, slot, y1)
            _set1(ax1_v, slot, x1)
            _set1(ay2_v, slot, y2)
            _set1(ax2_v, slot, x2)
            _set1(aar_v, slot, a2)
            _set1(ac_v, slot, c)
            _set1(os_v, slot, m)
            _set1(ol_v, slot, c)
            sbase = (slot // 16) * 16
            soff = slot - sbase
            for k, cv in enumerate((y1, x1, y2, x2)):
                blk = ob_v[k, pl.ds(sbase, 16)]
                ob_v[k, pl.ds(sbase, 16)] = jnp.where(lane == soff, cv, blk)

            m2, seg2 = best()
            return cnt + accept.astype(jnp.int32), m2, seg2

        cnt, _, _ = lax.while_loop(cond, body, (jnp.int32(0), m0, seg0))

        ov_v[...] = jnp.where(lane == 0, cnt, 0)
        for k in range(4):
            pltpu.sync_copy(ob_v.at[k], out_b_hbm.at[k])
        pltpu.sync_copy(os_v, out_s_hbm)
        pltpu.sync_copy(ol_v, out_l_hbm)
        pltpu.sync_copy(ov_v, out_v_hbm)


@jax.jit
def kernel(boxes, scores):
    sflat = jnp.pad(scores[0].reshape(-1), (0, _TOT - _N * _C))
    boxes4 = jnp.pad(boxes[0], ((0, _NPAD - _N), (0, 0))).T.reshape(-1)

    k = pl.kernel(
        _nms_body,
        out_type=(
            jax.ShapeDtypeStruct((4, 128), jnp.float32),
            jax.ShapeDtypeStruct((128,), jnp.float32),
            jax.ShapeDtypeStruct((128,), jnp.int32),
            jax.ShapeDtypeStruct((16,), jnp.int32),
        ),
        mesh=_get_mesh(),
        scratch_types=[
            pltpu.VMEM_SHARED((_TOT,), jnp.float32),
            pltpu.VMEM_SHARED((_NSEG,), jnp.float32),
            pltpu.VMEM_SHARED((_NSEG,), jnp.int32),
            pltpu.VMEM_SHARED((_NSEG,), jnp.float32),
            pltpu.VMEM((_CHUNK,), jnp.float32),
            pltpu.VMEM((_SLOT_PER_W,), jnp.float32),
            pltpu.VMEM((_SLOT_PER_W,), jnp.int32),
            pltpu.VMEM((_SLOT_PER_W,), jnp.float32),
            pltpu.VMEM((_NSEG,), jnp.float32),
            pltpu.VMEM((_NSEG // 16,), jnp.float32),
            pltpu.VMEM((_NSEG,), jnp.int32),
            pltpu.VMEM((_NSEG,), jnp.float32),
            pltpu.VMEM((_NSEG,), jnp.int32),
            pltpu.VMEM((_NSEG,), jnp.float32),
            pltpu.VMEM((_SEG,), jnp.float32),
            pltpu.VMEM((4 * _NPAD,), jnp.float32),
            pltpu.VMEM((128,), jnp.float32),
            pltpu.VMEM((128,), jnp.float32),
            pltpu.VMEM((128,), jnp.float32),
            pltpu.VMEM((128,), jnp.float32),
            pltpu.VMEM((128,), jnp.float32),
            pltpu.VMEM((128,), jnp.int32),
            pltpu.VMEM((4, 128), jnp.float32),
            pltpu.VMEM((128,), jnp.float32),
            pltpu.VMEM((128,), jnp.int32),
            pltpu.VMEM((16,), jnp.int32),
        ],
        compiler_params=pltpu.CompilerParams(needs_layout_passes=False),
    )
    ob4, osc, ol, ov = k(sflat, boxes4)

    return (ob4.T[:MAX_DETECTIONS].reshape(1, MAX_DETECTIONS, 4),
            osc[:MAX_DETECTIONS].reshape(1, MAX_DETECTIONS),
            ol[:MAX_DETECTIONS].reshape(1, MAX_DETECTIONS),
            ov[0].reshape(1))


# speedup vs baseline: 54.3149x; 1.1044x over previous
import functools

import jax
import jax.numpy as jnp
from jax import lax
from jax.experimental import pallas as pl
from jax.experimental.pallas import tpu as pltpu
from jax.experimental.pallas import tpu_sc as plsc

SCORE_THRESHOLD = 0.05
IOU_THRESHOLD = 0.5
MAX_DETECTIONS = 100
NEG = -1e30

_N = 5000
_C = 80
_NPAD = 5120
_SEG = 256
_W = 16
_SEG_PER_W = 100
_SLOT_PER_W = 112
_NSEG = _W * _SLOT_PER_W
_CHUNK = _SEG_PER_W * _SEG
_TOT = _W * _CHUNK
_LASTR = _N * _C - (_W - 1) * _CHUNK
_BIG = 2**30
_UNK = -1


@functools.lru_cache(maxsize=None)
def _get_mesh():
    return plsc.VectorSubcoreMesh(core_axis_name="c", subcore_axis_name="s",
                                  num_cores=2, num_subcores=16)


def _set1(ref, i, val):
    lane = lax.iota(jnp.int32, 16)
    base = (i // 16) * 16
    blk = ref[pl.ds(base, 16)]
    ref[pl.ds(base, 16)] = jnp.where(lane == i - base, val, blk)


def _get1(ref, i):
    idx = jnp.full((16,), 0, jnp.int32) + i
    return plsc.load_gather(ref, [idx])[0]


def _nms_body(scores_hbm, boxes4_hbm,
              out_b_hbm, out_s_hbm, out_l_hbm, out_v_hbm,
              spm_sc, spm_sm, spm_p1, spm_nx,
              chunk_v, smax_v, p1st_v, nxst_v,
              sm_v, l2_v, cp_v, lv_v, lp_v, nx_v, seg_v, bco_v,
              ay1_v, ax1_v, ay2_v, ax2_v, aar_v, ac_v,
              ob_v, os_v, ol_v, ov_v):
    cid = lax.axis_index("c")
    sid = lax.axis_index("s")
    lane = lax.iota(jnp.int32, 16)

    @pl.when(cid == 0)
    def _phase1():
        base = sid * _CHUNK

        @pl.when(sid < _W - 1)
        def _load_full():
            pltpu.sync_copy(scores_hbm.at[pl.ds(base, _CHUNK)], chunk_v)

        @pl.when(sid == _W - 1)
        def _load_tail():
            pltpu.sync_copy(scores_hbm.at[pl.ds(base, _LASTR)],
                            chunk_v.at[pl.ds(0, _LASTR)])

        @pl.when(sid == _W - 1)
        def _fill_tail():
            zf16 = jnp.zeros((16,), jnp.float32)

            def fill(j, _):
                for u in range(8):
                    chunk_v[pl.ds(_LASTR + (j * 8 + u) * 16, 16)] = zf16
                return 0

            lax.fori_loop(0, (_CHUNK - _LASTR) // 128, fill, 0)

        pltpu.sync_copy(chunk_v, spm_sc.at[pl.ds(base, _CHUNK)])

        def per_group(sg, _):
            slot = sg * 16 + lane
            real = slot < _SEG_PER_W
            lbase = jnp.where(real, slot * _SEG, 0)
            fbase = base + lbase

            def step(k16, carry):
                mv, pm, sv = carry
                lb = lbase + k16 * 16
                fb = fbase + k16 * 16
                for k2 in range(16):
                    g = plsc.load_gather(chunk_v, [lb + k2])
                    gt = g > mv
                    sv = jnp.where(gt, mv, jnp.maximum(sv, g))
                    pm = jnp.where(gt, fb + k2, pm)
                    mv = jnp.maximum(mv, g)
                return mv, pm, sv

            mv, pm, sv = lax.fori_loop(
                0, _SEG // 16, step,
                (jnp.full((16,), NEG, jnp.float32),
                 jnp.zeros((16,), jnp.int32),
                 jnp.full((16,), NEG, jnp.float32)))
            smax_v[pl.ds(sg * 16, 16)] = jnp.where(real, mv, NEG)
            p1st_v[pl.ds(sg * 16, 16)] = pm
            nxst_v[pl.ds(sg * 16, 16)] = jnp.where(real, sv, NEG)
            return 0

        lax.fori_loop(0, _SLOT_PER_W // 16, per_group, 0)
        off = sid * _SLOT_PER_W
        pltpu.sync_copy(smax_v, spm_sm.at[pl.ds(off, _SLOT_PER_W)])
        pltpu.sync_copy(p1st_v, spm_p1.at[pl.ds(off, _SLOT_PER_W)])
        pltpu.sync_copy(nxst_v, spm_nx.at[pl.ds(off, _SLOT_PER_W)])

    plsc.subcore_barrier()

    @pl.when((cid == 0) & (sid == 0))
    def _phase2():
        pltpu.sync_copy(spm_sm, sm_v)
        pltpu.sync_copy(spm_p1, cp_v)
        pltpu.sync_copy(spm_nx, nx_v)
        pltpu.sync_copy(boxes4_hbm, bco_v)

        def init_l2(gg, _):
            gidx = (gg * 16 + lane) * 16

            mvv = jnp.full((16,), NEG, jnp.float32)
            for k in range(16):
                mvv = jnp.maximum(mvv, plsc.load_gather(sm_v, [gidx + k]))
            l2_v[pl.ds(gg * 16, 16)] = mvv
            return 0

        lax.fori_loop(0, _NSEG // 256, init_l2, 0)

        zf = jnp.zeros((16,), jnp.float32)
        zi = jnp.zeros((16,), jnp.int32)

        def init_out(j, _):
            os_v[pl.ds(j * 16, 16)] = zf
            ol_v[pl.ds(j * 16, 16)] = zi
            ay1_v[pl.ds(j * 16, 16)] = zf
            ax1_v[pl.ds(j * 16, 16)] = zf
            ay2_v[pl.ds(j * 16, 16)] = zf
            ax2_v[pl.ds(j * 16, 16)] = zf
            aar_v[pl.ds(j * 16, 16)] = zf
            ac_v[pl.ds(j * 16, 16)] = jnp.full((16,), -1, jnp.int32)
            lv_v[pl.ds(j * 16, 16)] = zf
            lp_v[pl.ds(j * 16, 16)] = zi
            for k in range(4):
                ob_v[k, pl.ds(j * 16, 16)] = zf
            return 0

        lax.fori_loop(0, 8, init_out, 0)

        def zero_lvlp(j, _):
            lv_v[pl.ds(j * 16, 16)] = zf
            lp_v[pl.ds(j * 16, 16)] = zi
            return 0

        lax.fori_loop(8, _NSEG // 16, zero_lvlp, 0)

        def best():
            mvv = l2_v[pl.ds(0, 16)]
            for g in range(1, 7):
                mvv = jnp.maximum(mvv, l2_v[pl.ds(g * 16, 16)])
            m = jnp.max(mvv)

            gm = jnp.full((16,), _BIG, jnp.int32)
            for g in range(7):
                blk = l2_v[pl.ds(g * 16, 16)]
                gm = jnp.minimum(gm, jnp.where(blk == m, g * 16 + lane,
                                               _BIG))
            g_star = jnp.min(gm)
            blk = sm_v[pl.ds(g_star * 16, 16)]
            seg = jnp.min(jnp.where(blk == m, g_star * 16 + lane, _BIG))
            return m, seg

        m0, seg0 = best()

        def cond(carry):
            cnt, m, _ = carry
            return (cnt < MAX_DETECTIONS) & (m > SCORE_THRESHOLD)

        def body(carry):
            cnt, m, seg = carry
            cp = _get1(cp_v, seg)

            def fast(_):
                return cp, _get1(nx_v, seg)

            def slow(_):
                lv = _get1(lv_v, seg)
                lp = _get1(lp_v, seg)
                ebase = ((seg // _SLOT_PER_W) * _CHUNK
                         + (seg % _SLOT_PER_W) * _SEG)
                pltpu.sync_copy(spm_sc.at[pl.ds(ebase, _SEG)], seg_v)

                def find(j, pm):
                    v = seg_v[pl.ds(j * 16, 16)]
                    pos = ebase + j * 16 + lane
                    elig = (v < lv) | ((v == lv) & (pos > lp))
                    return jnp.minimum(pm, jnp.where(elig & (v == m), pos,
                                                     _BIG))

                p_cur = jnp.min(lax.fori_loop(
                    0, _SEG // 16, find, jnp.full((16,), _BIG, jnp.int32)))

                def nxt(j, nv):
                    v = seg_v[pl.ds(j * 16, 16)]
                    pos = ebase + j * 16 + lane
                    elig = ((v < lv) | ((v == lv) & (pos > lp))) & \
                           ((v < m) | ((v == m) & (pos > p_cur)))
                    return jnp.maximum(nv, jnp.where(elig, v, NEG))

                v_next = jnp.max(lax.fori_loop(
                    0, _SEG // 16, nxt,
                    jnp.full((16,), NEG, jnp.float32)))
                return p_cur, v_next

            p_cur, v_next = lax.cond(cp != _UNK, fast, slow, 0)

            _set1(lv_v, seg, m)
            _set1(lp_v, seg, p_cur)
            _set1(cp_v, seg, _UNK)
            _set1(sm_v, seg, v_next)
            g = seg // 16
            _set1(l2_v, g, jnp.max(sm_v[pl.ds(g * 16, 16)]))

            b = p_cur // _C
            c = p_cur % _C

            gidx = b * 4 + jnp.where(lane < 4, lane, 0)
            bco = plsc.load_gather(bco_v, [gidx])
            y1 = bco[0]
            x1 = bco[1]
            y2 = bco[2]
            x2 = bco[3]
            a2 = (y2 - y1) * (x2 - x1)

            def chk(k, nv):
                qy1 = jnp.maximum(y1, ay1_v[pl.ds(k * 16, 16)])
                qx1 = jnp.maximum(x1, ax1_v[pl.ds(k * 16, 16)])
                qy2 = jnp.minimum(y2, ay2_v[pl.ds(k * 16, 16)])
                qx2 = jnp.minimum(x2, ax2_v[pl.ds(k * 16, 16)])
                inter = (jnp.maximum(qy2 - qy1, 0.0)
                         * jnp.maximum(qx2 - qx1, 0.0))
                denom = jnp.maximum(aar_v[pl.ds(k * 16, 16)] + a2 - inter,
                                    1e-9)
                iou = inter / denom
                hit = (iou > IOU_THRESHOLD) & (ac_v[pl.ds(k * 16, 16)] == c)
                return nv + hit.astype(jnp.int32)

            nviol = lax.fori_loop(0, (cnt + 15) // 16, chk,
                                  jnp.zeros((16,), jnp.int32))
            accept = jnp.sum(nviol) == 0
            slot = jnp.where(accept, cnt, 127)

            _set1(ay1_v, slot, y1)
            _set1(ax1_v, slot, x1)
            _set1(ay2_v, slot, y2)
            _set1(ax2_v, slot, x2)
            _set1(aar_v, slot, a2)
            _set1(ac_v, slot, c)
            _set1(os_v, slot, m)
            _set1(ol_v, slot, c)
            sbase = (slot // 16) * 16
            soff = slot - sbase
            for k, cv in enumerate((y1, x1, y2, x2)):
                blk = ob_v[k, pl.ds(sbase, 16)]
                ob_v[k, pl.ds(sbase, 16)] = jnp.where(lane == soff, cv, blk)

            m2, seg2 = best()
            return cnt + accept.astype(jnp.int32), m2, seg2

        cnt, _, _ = lax.while_loop(cond, body, (jnp.int32(0), m0, seg0))

        ov_v[...] = jnp.where(lane == 0, cnt, 0)
        for k in range(4):
            pltpu.sync_copy(ob_v.at[k], out_b_hbm.at[k])
        pltpu.sync_copy(os_v, out_s_hbm)
        pltpu.sync_copy(ol_v, out_l_hbm)
        pltpu.sync_copy(ov_v, out_v_hbm)


@jax.jit
def kernel(boxes, scores):
    sflat = scores[0].reshape(-1)
    boxes4 = boxes[0].reshape(-1)

    k = pl.kernel(
        _nms_body,
        out_type=(
            jax.ShapeDtypeStruct((4, 128), jnp.float32),
            jax.ShapeDtypeStruct((128,), jnp.float32),
            jax.ShapeDtypeStruct((128,), jnp.int32),
            jax.ShapeDtypeStruct((16,), jnp.int32),
        ),
        mesh=_get_mesh(),
        scratch_types=[
            pltpu.VMEM_SHARED((_TOT,), jnp.float32),
            pltpu.VMEM_SHARED((_NSEG,), jnp.float32),
            pltpu.VMEM_SHARED((_NSEG,), jnp.int32),
            pltpu.VMEM_SHARED((_NSEG,), jnp.float32),
            pltpu.VMEM((_CHUNK,), jnp.float32),
            pltpu.VMEM((_SLOT_PER_W,), jnp.float32),
            pltpu.VMEM((_SLOT_PER_W,), jnp.int32),
            pltpu.VMEM((_SLOT_PER_W,), jnp.float32),
            pltpu.VMEM((_NSEG,), jnp.float32),
            pltpu.VMEM((_NSEG // 16,), jnp.float32),
            pltpu.VMEM((_NSEG,), jnp.int32),
            pltpu.VMEM((_NSEG,), jnp.float32),
            pltpu.VMEM((_NSEG,), jnp.int32),
            pltpu.VMEM((_NSEG,), jnp.float32),
            pltpu.VMEM((_SEG,), jnp.float32),
            pltpu.VMEM((4 * _N,), jnp.float32),
            pltpu.VMEM((128,), jnp.float32),
            pltpu.VMEM((128,), jnp.float32),
            pltpu.VMEM((128,), jnp.float32),
            pltpu.VMEM((128,), jnp.float32),
            pltpu.VMEM((128,), jnp.float32),
            pltpu.VMEM((128,), jnp.int32),
            pltpu.VMEM((4, 128), jnp.float32),
            pltpu.VMEM((128,), jnp.float32),
            pltpu.VMEM((128,), jnp.int32),
            pltpu.VMEM((16,), jnp.int32),
        ],
        compiler_params=pltpu.CompilerParams(needs_layout_passes=False,
                                     skip_device_barrier=True),
    )
    ob4, osc, ol, ov = k(sflat, boxes4)

    return (ob4.T[:MAX_DETECTIONS].reshape(1, MAX_DETECTIONS, 4),
            osc[:MAX_DETECTIONS].reshape(1, MAX_DETECTIONS),
            ol[:MAX_DETECTIONS].reshape(1, MAX_DETECTIONS),
            ov[0].reshape(1))
